# baseline (speedup 1.0000x reference)
import functools

import jax
import jax.numpy as jnp
from jax import lax
from jax.experimental import pallas as pl
from jax.experimental.pallas import tpu as pltpu
from jax.experimental.pallas import tpu_sc as plsc

_B = 1024
_INPUT = 511
_D = 128
_NVR = _D // 16
_NC = 2
_NS = 16
_P0 = 18
_P1 = 14
_CB = 128
_NBB = _B // _CB
_NSLOT = 4
_PF = 32


def _sc_body(xt_ref, tok_ref, seg_ref, pos_ref, out_ref,
             idx_v, bias_v, seg_v, rows, gsems, ssems):
    c = lax.axis_index("c")
    s = lax.axis_index("s")
    is0 = c == 0
    npos = jnp.where(is0, _P0, _P1)
    p_base = jnp.where(is0, s * _P0, 16 * _P0 + s * _P1)
    pf_base = pl.multiple_of(
        jnp.minimum((p_base >> 3) << 3, _INPUT + 1 - _PF), 8)
    off = p_base - pf_base

    pltpu.sync_copy(xt_ref.at[pl.ds(pf_base, _PF)], idx_v)

    pltpu.sync_copy(pos_ref.at[pl.ds(pf_base, _PF)], bias_v)
    pltpu.sync_copy(seg_ref, seg_v)
    for i in range(_PF):
        seg_row = jnp.where(pf_base + i >= 256, 1, 0)
        for j in range(_NVR):
            sl = pl.ds(16 * j, 16)
            bias_v[i, sl] = bias_v[i, sl] + seg_v[seg_row, sl]

    def start_gather(t, sl_):
        pltpu.async_copy(tok_ref.at[idx_v.at[off + (t >> 3), t & 7]],
                         rows[sl_], gsems[sl_])

    def wait_gather(sl_):
        pltpu.make_async_copy(tok_ref.at[pl.ds(0, _CB)], rows[sl_],
                              gsems[sl_]).wait()

    def start_store(t, sl_):
        pi = t >> 3
        b0 = (t & 7) * _CB
        p = p_base + pi

        @pl.when(p < _INPUT)
        def _():
            pltpu.async_copy(rows[sl_], out_ref.at[p, pl.ds(b0, _CB)],
                             ssems[sl_])

    def wait_store(t, sl_):
        p = p_base + (t >> 3)

        @pl.when(p < _INPUT)
        def _():
            pltpu.make_async_copy(tok_ref.at[pl.ds(0, _CB)], rows[sl_],
                                  ssems[sl_]).wait()

    def add_bias(t, sl_):
        row_v = rows[sl_]
        pi = off + (t >> 3)
        bias_regs = [bias_v[pi, pl.ds(16 * j, 16)] for j in range(_NVR)]

        def body(i, carry):
            for u in range(4):
                r = 4 * i + u
                for j in range(_NVR):
                    plsc.addupdate(row_v.at[r, pl.ds(16 * j, 16)],
                                   bias_regs[j])
            return carry

        lax.fori_loop(0, _CB // 4, body, 0)

    nchunk = npos * _NBB
    start_gather(0, 0)
    start_gather(1, 1)

    def ring_body(i, carry):
        for u in range(_NSLOT):
            t = _NSLOT * i + u
            s2 = (u + 2) % _NSLOT

            @pl.when(t >= 2)
            def _():
                wait_store(t - 2, s2)

            @pl.when(t + 2 < nchunk)
            def _():
                start_gather(t + 2, s2)

            wait_gather(u)
            add_bias(t, u)
            start_store(t, u)
        return carry

    lax.fori_loop(0, nchunk // _NSLOT, ring_body, 0)

    wait_store(nchunk - 2, (_NSLOT - 2) % _NSLOT)
    wait_store(nchunk - 1, (_NSLOT - 1) % _NSLOT)


@jax.jit
def _run(x_t3, token_table, segment_table, position_table):
    mesh = plsc.VectorSubcoreMesh(core_axis_name="c", subcore_axis_name="s")
    fn = functools.partial(
        pl.kernel,
        mesh=mesh,
        out_type=jax.ShapeDtypeStruct((_INPUT, _B, _D), jnp.float32),
        scratch_types=[
            pltpu.VMEM((_PF, _NBB, _CB), jnp.int32),
            pltpu.VMEM((_PF, _D), jnp.float32),
            pltpu.VMEM((2, _D), jnp.float32),
            [pltpu.VMEM((_CB, _D), jnp.float32) for _ in range(_NSLOT)],
            [pltpu.SemaphoreType.DMA for _ in range(_NSLOT)],
            [pltpu.SemaphoreType.DMA for _ in range(_NSLOT)],
        ],
    )(_sc_body)
    return fn(x_t3, token_table, segment_table, position_table)


def kernel(x, token_table, segment_table, position_table):
    x_pad = jnp.pad(x, ((0, 0), (0, 1)))
    x_t3 = x_pad.T.reshape(_INPUT + 1, _NBB, _CB)
    out_t = _run(x_t3, token_table, segment_table, position_table)
    return out_t.transpose(1, 0, 2)


# speedup vs baseline: 1.0004x; 1.0004x over previous
import functools

import jax
import jax.numpy as jnp
from jax import lax
from jax.experimental import pallas as pl
from jax.experimental.pallas import tpu as pltpu
from jax.experimental.pallas import tpu_sc as plsc

_B = 1024
_INPUT = 511
_D = 128
_NVR = _D // 16
_NC = 2
_NS = 16
_P0 = 18
_P1 = 14
_CB = 128
_NBB = _B // _CB
_NSLOT = 4
_PF = 32


def _sc_body(xt_ref, tok_ref, seg_ref, pos_ref, out_ref,
             idx_v, bias_v, seg_v, rows, gsems, ssems):
    c = lax.axis_index("c")
    s = lax.axis_index("s")
    is0 = c == 0
    npos = jnp.where(is0, _P0, _P1)
    p_base = jnp.where(is0, s * _P0, 16 * _P0 + s * _P1)
    pf_base = pl.multiple_of(
        jnp.minimum((p_base >> 3) << 3, _INPUT + 1 - _PF), 8)
    off = p_base - pf_base

    pltpu.sync_copy(xt_ref.at[pl.ds(pf_base, _PF)], idx_v)

    pltpu.sync_copy(pos_ref.at[pl.ds(pf_base, _PF)], bias_v)
    pltpu.sync_copy(seg_ref, seg_v)
    for i in range(_PF):
        seg_row = jnp.where(pf_base + i >= 256, 1, 0)
        for j in range(_NVR):
            sl = pl.ds(16 * j, 16)
            bias_v[i, sl] = bias_v[i, sl] + seg_v[seg_row, sl]

    def start_gather(t, sl_):
        pi = off + (t >> 3)
        bb = t & 7
        h = _CB // 2
        pltpu.async_copy(tok_ref.at[idx_v.at[pi, bb, pl.ds(0, h)]],
                         rows[sl_].at[pl.ds(0, h)], gsems[sl_])
        pltpu.async_copy(tok_ref.at[idx_v.at[pi, bb, pl.ds(h, h)]],
                         rows[sl_].at[pl.ds(h, h)], gsems[sl_])

    def wait_gather(sl_):
        pltpu.make_async_copy(tok_ref.at[pl.ds(0, _CB)], rows[sl_],
                              gsems[sl_]).wait()

    def start_store(t, sl_):
        pi = t >> 3
        b0 = (t & 7) * _CB
        p = p_base + pi

        @pl.when(p < _INPUT)
        def _():
            pltpu.async_copy(rows[sl_], out_ref.at[p, pl.ds(b0, _CB)],
                             ssems[sl_])

    def wait_store(t, sl_):
        p = p_base + (t >> 3)

        @pl.when(p < _INPUT)
        def _():
            pltpu.make_async_copy(tok_ref.at[pl.ds(0, _CB)], rows[sl_],
                                  ssems[sl_]).wait()

    def add_bias(t, sl_):
        row_v = rows[sl_]
        pi = off + (t >> 3)
        bias_regs = [bias_v[pi, pl.ds(16 * j, 16)] for j in range(_NVR)]

        def body(i, carry):
            for u in range(4):
                r = 4 * i + u
                for j in range(_NVR):
                    plsc.addupdate(row_v.at[r, pl.ds(16 * j, 16)],
                                   bias_regs[j])
            return carry

        lax.fori_loop(0, _CB // 4, body, 0)

    nchunk = npos * _NBB
    start_gather(0, 0)
    start_gather(1, 1)

    def ring_body(i, carry):
        for u in range(_NSLOT):
            t = _NSLOT * i + u
            s2 = (u + 2) % _NSLOT

            @pl.when(t >= 2)
            def _():
                wait_store(t - 2, s2)

            @pl.when(t + 2 < nchunk)
            def _():
                start_gather(t + 2, s2)

            wait_gather(u)
            add_bias(t, u)
            start_store(t, u)
        return carry

    lax.fori_loop(0, nchunk // _NSLOT, ring_body, 0)

    wait_store(nchunk - 2, (_NSLOT - 2) % _NSLOT)
    wait_store(nchunk - 1, (_NSLOT - 1) % _NSLOT)


@jax.jit
def _run(x_t3, token_table, segment_table, position_table):
    mesh = plsc.VectorSubcoreMesh(core_axis_name="c", subcore_axis_name="s")
    fn = functools.partial(
        pl.kernel,
        mesh=mesh,
        out_type=jax.ShapeDtypeStruct((_INPUT, _B, _D), jnp.float32),
        scratch_types=[
            pltpu.VMEM((_PF, _NBB, _CB), jnp.int32),
            pltpu.VMEM((_PF, _D), jnp.float32),
            pltpu.VMEM((2, _D), jnp.float32),
            [pltpu.VMEM((_CB, _D), jnp.float32) for _ in range(_NSLOT)],
            [pltpu.SemaphoreType.DMA for _ in range(_NSLOT)],
            [pltpu.SemaphoreType.DMA for _ in range(_NSLOT)],
        ],
    )(_sc_body)
    return fn(x_t3, token_table, segment_table, position_table)


def kernel(x, token_table, segment_table, position_table):
    x_pad = jnp.pad(x, ((0, 0), (0, 1)))
    x_t3 = x_pad.T.reshape(_INPUT + 1, _NBB, _CB)
    out_t = _run(x_t3, token_table, segment_table, position_table)
    return out_t.transpose(1, 0, 2)


# speedup vs baseline: 1.0174x; 1.0170x over previous
import functools

import jax
import jax.numpy as jnp
from jax import lax
from jax.experimental import pallas as pl
from jax.experimental.pallas import tpu as pltpu
from jax.experimental.pallas import tpu_sc as plsc

_B = 1024
_INPUT = 511
_D = 128
_NVR = _D // 16
_NC = 2
_NS = 16
_PW = 16
_CB = 128
_NBB = _B // _CB
_NCHUNK = _PW * _NBB
_NSLOT = 4


def _sc_body(xt_ref, tok_ref, seg_ref, pos_ref, out_ref,
             idx_v, bias_v, seg_v, rows, gsems, ssems):
    w = lax.axis_index("s") * _NC + lax.axis_index("c")
    p_base = w * _PW

    pltpu.sync_copy(xt_ref.at[pl.ds(p_base, _PW)], idx_v)

    pltpu.sync_copy(pos_ref.at[pl.ds(p_base, _PW)], bias_v)
    pltpu.sync_copy(seg_ref, seg_v)
    for i in range(_PW):
        seg_row = jnp.where(p_base + i >= 256, 1, 0)
        for j in range(_NVR):
            sl = pl.ds(16 * j, 16)
            bias_v[i, sl] = bias_v[i, sl] + seg_v[seg_row, sl]

    def start_gather(t, sl_):
        pltpu.async_copy(tok_ref.at[idx_v.at[t >> 3, t & 7]],
                         rows[sl_], gsems[sl_])

    def wait_gather(sl_):
        pltpu.make_async_copy(tok_ref.at[pl.ds(0, _CB)], rows[sl_],
                              gsems[sl_]).wait()

    def start_store(t, sl_):
        pi = t >> 3
        b0 = (t & 7) * _CB
        p = p_base + pi

        @pl.when(p < _INPUT)
        def _():
            pltpu.async_copy(rows[sl_], out_ref.at[p, pl.ds(b0, _CB)],
                             ssems[sl_])

    def wait_store(t, sl_):
        p = p_base + (t >> 3)

        @pl.when(p < _INPUT)
        def _():
            pltpu.make_async_copy(tok_ref.at[pl.ds(0, _CB)], rows[sl_],
                                  ssems[sl_]).wait()

    def add_bias(t, sl_):
        row_v = rows[sl_]
        pi = t >> 3
        bias_regs = [bias_v[pi, pl.ds(16 * j, 16)] for j in range(_NVR)]

        def body(i, carry):
            for u in range(4):
                r = 4 * i + u
                for j in range(_NVR):
                    plsc.addupdate(row_v.at[r, pl.ds(16 * j, 16)],
                                   bias_regs[j])
            return carry

        lax.fori_loop(0, _CB // 4, body, 0)

    start_gather(0, 0)
    start_gather(1, 1)

    def ring_body(i, carry):
        for u in range(_NSLOT):
            t = _NSLOT * i + u
            s2 = (u + 2) % _NSLOT

            @pl.when(t >= 2)
            def _():
                wait_store(t - 2, s2)

            @pl.when(t + 2 < _NCHUNK)
            def _():
                start_gather(t + 2, s2)

            wait_gather(u)
            add_bias(t, u)
            start_store(t, u)
        return carry

    lax.fori_loop(0, _NCHUNK // _NSLOT, ring_body, 0)

    wait_store(_NCHUNK - 2, (_NSLOT - 2) % _NSLOT)
    wait_store(_NCHUNK - 1, (_NSLOT - 1) % _NSLOT)


@jax.jit
def _run(x_t3, token_table, segment_table, position_table):
    mesh = plsc.VectorSubcoreMesh(core_axis_name="c", subcore_axis_name="s")
    fn = functools.partial(
        pl.kernel,
        mesh=mesh,
        out_type=jax.ShapeDtypeStruct((_INPUT, _B, _D), jnp.float32),
        scratch_types=[
            pltpu.VMEM((_PW, _NBB, _CB), jnp.int32),
            pltpu.VMEM((_PW, _D), jnp.float32),
            pltpu.VMEM((2, _D), jnp.float32),
            [pltpu.VMEM((_CB, _D), jnp.float32) for _ in range(_NSLOT)],
            [pltpu.SemaphoreType.DMA for _ in range(_NSLOT)],
            [pltpu.SemaphoreType.DMA for _ in range(_NSLOT)],
        ],
    )(_sc_body)
    return fn(x_t3, token_table, segment_table, position_table)


def kernel(x, token_table, segment_table, position_table):
    x_pad = jnp.pad(x, ((0, 0), (0, 1)))
    x_t3 = x_pad.T.reshape(_INPUT + 1, _NBB, _CB)
    out_t = _run(x_t3, token_table, segment_table, position_table)
    return out_t.transpose(1, 0, 2)
